# Initial kernel scaffold; baseline (speedup 1.0000x reference)
import jax
import jax.numpy as jnp
from jax.experimental import pallas as pl


def kernel(hidden_states, gate_w, w1, w3, w2):
    raise NotImplementedError("write your pallas kernel here")


# speedup vs baseline: 1.1450x; 1.1450x over previous
import functools

import jax
import jax.numpy as jnp
from jax.experimental import pallas as pl
from jax.experimental.pallas import tpu as pltpu

B, S, D = 1, 2048, 1024
E, K, F = 8, 2, 2048
T = B * S

BS = 256
FB = 1024
NF = F // FB
NB = (T * K) // BS + E
PADDED = NB * BS


def _gate_body(x_ref, gw_ref, i1_ref, i2_ref, w1_ref, w2_ref):
    x = x_ref[...]
    gw = gw_ref[...]
    logits = jax.lax.dot_general(
        x, gw, (((1,), (1,)), ((), ())), preferred_element_type=jnp.float32
    )
    iota = jax.lax.broadcasted_iota(jnp.int32, logits.shape, 1)
    m1 = jnp.max(logits, axis=1, keepdims=True)
    i1 = jnp.min(jnp.where(logits == m1, iota, E), axis=1, keepdims=True)
    masked = jnp.where(iota == i1, -jnp.inf, logits)
    m2 = jnp.max(masked, axis=1, keepdims=True)
    i2 = jnp.min(jnp.where(masked == m2, iota, E), axis=1, keepdims=True)
    wa = jax.nn.sigmoid(m1 - m2)
    i1_ref[...] = i1
    i2_ref[...] = i2
    w1_ref[...] = wa
    w2_ref[...] = 1.0 - wa


def _moe_body(be_ref, ids_ref, w1_ref, w3_ref, w2_ref, x_ref, sw_ref,
              out_ref, xb_ref, acc_ref):
    b = pl.program_id(0)
    f = pl.program_id(1)

    @pl.when(jnp.logical_and(b == 0, f == 0))
    def _():
        out_ref[...] = jnp.zeros_like(out_ref)

    base = b * BS

    @pl.when(f == 0)
    def _():
        def gather(i, _):
            t = ids_ref[base + i]
            xb_ref[pl.ds(i, 1), :] = x_ref[pl.ds(t, 1), :]
            return 0
        jax.lax.fori_loop(0, BS, gather, 0)

    xb = xb_ref[...]
    h1 = jax.lax.dot_general(
        xb, w1_ref[0], (((1,), (1,)), ((), ())), preferred_element_type=jnp.float32
    )
    h3 = jax.lax.dot_general(
        xb, w3_ref[0], (((1,), (1,)), ((), ())), preferred_element_type=jnp.float32
    )
    h = h1 * jax.nn.sigmoid(h1) * h3
    y = jax.lax.dot_general(
        h, w2_ref[0], (((1,), (1,)), ((), ())), preferred_element_type=jnp.float32
    )

    @pl.when(f == 0)
    def _():
        acc_ref[...] = y

    @pl.when(f > 0)
    def _():
        acc_ref[...] += y

    @pl.when(f == NF - 1)
    def _():
        acc_ref[...] = acc_ref[...] * sw_ref[...]

        def scatter(i, _):
            t = ids_ref[base + i]
            out_ref[pl.ds(t, 1), :] += acc_ref[pl.ds(i, 1), :]
            return 0
        jax.lax.fori_loop(0, BS, scatter, 0)


@jax.jit
def kernel(hidden_states, gate_w, w1, w3, w2):
    orig_shape = hidden_states.shape
    x = hidden_states.reshape(T, D)

    i1, i2, wa, wb = pl.pallas_call(
        _gate_body,
        out_shape=(
            jax.ShapeDtypeStruct((T, 1), jnp.int32),
            jax.ShapeDtypeStruct((T, 1), jnp.int32),
            jax.ShapeDtypeStruct((T, 1), jnp.float32),
            jax.ShapeDtypeStruct((T, 1), jnp.float32),
        ),
    )(x, gate_w)

    flat_e = jnp.concatenate([i1, i2], axis=1).reshape(-1)
    flat_w = jnp.concatenate([wa, wb], axis=1).reshape(-1)
    oh = (flat_e[:, None] == jnp.arange(E)[None, :]).astype(jnp.int32)
    counts = jnp.sum(oh, axis=0)
    padded = ((counts + BS - 1) // BS) * BS
    offs = jnp.concatenate([jnp.zeros(1, jnp.int32),
                            jnp.cumsum(padded)[:-1].astype(jnp.int32)])
    rank = jnp.cumsum(oh, axis=0) - 1
    my_rank = jnp.take_along_axis(rank, flat_e[:, None], axis=1)[:, 0]
    pos = offs[flat_e] + my_rank
    sort_ids = jnp.zeros(PADDED, jnp.int32).at[pos].set(
        jnp.arange(T * K, dtype=jnp.int32) // K)
    slot_w = jnp.zeros(PADDED, jnp.float32).at[pos].set(flat_w)
    block_expert = jnp.sum(
        (jnp.arange(NB, dtype=jnp.int32)[:, None] * BS) >= offs[None, :],
        axis=1, dtype=jnp.int32) - 1

    grid_spec = pltpu.PrefetchScalarGridSpec(
        num_scalar_prefetch=2,
        grid=(NB, NF),
        in_specs=[
            pl.BlockSpec((1, FB, D), lambda b, f, be, ids: (be[b], f, 0)),
            pl.BlockSpec((1, FB, D), lambda b, f, be, ids: (be[b], f, 0)),
            pl.BlockSpec((1, D, FB), lambda b, f, be, ids: (be[b], 0, f)),
            pl.BlockSpec((T, D), lambda b, f, be, ids: (0, 0)),
            pl.BlockSpec((BS, 1), lambda b, f, be, ids: (b, 0)),
        ],
        out_specs=pl.BlockSpec((T, D), lambda b, f, be, ids: (0, 0)),
        scratch_shapes=[
            pltpu.VMEM((BS, D), jnp.float32),
            pltpu.VMEM((BS, D), jnp.float32),
        ],
    )
    y = pl.pallas_call(
        _moe_body,
        grid_spec=grid_spec,
        out_shape=jax.ShapeDtypeStruct((T, D), jnp.float32),
        compiler_params=pltpu.CompilerParams(
            dimension_semantics=("arbitrary", "arbitrary"),
        ),
    )(block_expert, sort_ids, w1, w3, w2, x, slot_w[:, None])
    return y.reshape(orig_shape)
